# Initial kernel scaffold; baseline (speedup 1.0000x reference)
import jax
import jax.numpy as jnp
from jax.experimental import pallas as pl


def kernel(x, edge_index, batchidxs, W_gat, a_src, a_dst, b_gat, eps, W1, b1, W2, b2, Wg1, bg1, Wg2, bg2, Wf1, bf1, Wf2, bf2):
    raise NotImplementedError("write your pallas kernel here")


# speedup vs baseline: 12.6510x; 12.6510x over previous
import functools

import jax
import jax.numpy as jnp
from jax import lax
from jax.experimental import pallas as pl
from jax.experimental.pallas import tpu as pltpu
from jax.experimental.pallas import tpu_sc as plsc

N = 10000
E = 320000
B = 100
NF = 128
NG = 16
NH = 256
D = NF + NG
DP = 160
NPROP = 2

NC = 2
NS = 16
NW = NC * NS
EPW = E // NW
CH = 80
NSL = N // NS

_f32 = jnp.float32


def _mesh():
    return plsc.VectorSubcoreMesh(
        core_axis_name="c", subcore_axis_name="s", num_cores=NC, num_subcores=NS
    )


@functools.partial(
    pl.kernel,
    out_type=jax.ShapeDtypeStruct((NC * N, DP), _f32),
    mesh=_mesh(),
    scratch_types=[
        pltpu.VMEM((CH,), jnp.int32),
        pltpu.VMEM((CH,), jnp.int32),
        pltpu.VMEM((CH,), _f32),
        pltpu.VMEM((CH,), _f32),
        pltpu.VMEM((CH,), _f32),
        pltpu.VMEM((CH, DP), _f32),
        pltpu.VMEM_SHARED((N, DP), _f32),
        pltpu.SemaphoreType.DMA,
        pltpu.SemaphoreType.DMA,
    ],
    compiler_params=pltpu.CompilerParams(
        use_tc_tiling_on_sc=False, needs_layout_passes=False),
)
def _gat_edges(hp_hbm, es_hbm, ed_hbm, src_hbm, dst_hbm, z_hbm, out_hbm,
               idx_s, idx_d, sv, dv, w, rows, acc, sem1, sem2):
    c = lax.axis_index("c")
    s = lax.axis_index("s")
    wid = c * NS + s
    ebase = wid * EPW

    pltpu.sync_copy(z_hbm, acc.at[pl.ds(s * NSL, NSL)])
    plsc.subcore_barrier()

    def chunk_body(i, _):
        base = ebase + i * CH
        pltpu.sync_copy(src_hbm.at[pl.ds(base, CH)], idx_s)
        pltpu.sync_copy(dst_hbm.at[pl.ds(base, CH)], idx_d)
        cpr = pltpu.async_copy(hp_hbm.at[idx_s], rows, sem1)
        cps = pltpu.async_copy(es_hbm.at[idx_s], sv, sem2)
        cpd = pltpu.async_copy(ed_hbm.at[idx_d], dv, sem2)
        cps.wait()
        cpd.wait()

        def grp(j, _):
            ev = sv[pl.ds(j * 16, 16)] + dv[pl.ds(j * 16, 16)]
            ev = jnp.where(ev >= 0.0, ev, 0.2 * ev)
            w[pl.ds(j * 16, 16)] = jnp.exp(ev)
            return 0

        lax.fori_loop(0, CH // 16, grp, 0)
        cpr.wait()

        def rw(g, _):
            wv16 = w[pl.ds(g * 16, 16)]
            for l in range(16):
                wv = wv16[l]
                j = g * 16 + l
                for k in range(DP // 16):
                    rows[j, pl.ds(k * 16, 16)] = rows[j, pl.ds(k * 16, 16)] * wv
            return 0

        lax.fori_loop(0, CH // 16, rw, 0)
        pltpu.sync_copy(rows, acc.at[idx_d], add=True)
        return 0

    lax.fori_loop(0, EPW // CH, chunk_body, 0)

    plsc.subcore_barrier()
    pltpu.sync_copy(acc.at[pl.ds(s * NSL, NSL)],
                    out_hbm.at[pl.ds(c * N + s * NSL, NSL)])


@functools.partial(
    pl.kernel,
    out_type=jax.ShapeDtypeStruct((NC * N, D), _f32),
    mesh=_mesh(),
    scratch_types=[
        pltpu.VMEM((CH,), jnp.int32),
        pltpu.VMEM((CH,), jnp.int32),
        pltpu.VMEM((CH, D), _f32),
        pltpu.VMEM_SHARED((N, D), _f32),
        pltpu.SemaphoreType.DMA,
    ],
    compiler_params=pltpu.CompilerParams(
        use_tc_tiling_on_sc=False, needs_layout_passes=False),
)
def _gin_edges(x_hbm, src_hbm, dst_hbm, z_hbm, out_hbm,
               idx_s, idx_d, rows, acc, sem):
    c = lax.axis_index("c")
    s = lax.axis_index("s")
    wid = c * NS + s
    ebase = wid * EPW

    pltpu.sync_copy(z_hbm, acc.at[pl.ds(s * NSL, NSL)])
    plsc.subcore_barrier()

    def chunk_body(i, _):
        base = ebase + i * CH
        pltpu.sync_copy(src_hbm.at[pl.ds(base, CH)], idx_s)
        pltpu.sync_copy(dst_hbm.at[pl.ds(base, CH)], idx_d)
        pltpu.async_copy(x_hbm.at[idx_s], rows, sem).wait()
        pltpu.sync_copy(rows, acc.at[idx_d], add=True)
        return 0

    lax.fori_loop(0, EPW // CH, chunk_body, 0)

    plsc.subcore_barrier()
    pltpu.sync_copy(acc.at[pl.ds(s * NSL, NSL)],
                    out_hbm.at[pl.ds(c * N + s * NSL, NSL)])


def _dot(a, b):
    return jnp.dot(a, b, precision=lax.Precision.HIGHEST,
                   preferred_element_type=_f32)


def _dot_hi(a, b):
    return jnp.dot(a, b, precision=lax.Precision.HIGHEST,
                   preferred_element_type=_f32)


def _hlvs_body(x_ref, pm_ref, wg1_ref, bg1_ref, wg2_ref, bg2_ref, hl_ref):
    mean = _dot_hi(pm_ref[...], x_ref[...])
    g = jnp.maximum(_dot(mean, wg1_ref[...]) + bg1_ref[...], 0.0)
    hl_ref[...] = _dot(g, wg2_ref[...]) + bg2_ref[...]


def _proj_body(x_ref, rm_ref, hl_ref, wgat_ref, av_ref,
               hp_ref, es_ref, ed_ref, wself_ref):
    x = x_ref[...]
    hln = _dot_hi(rm_ref[...], hl_ref[...])
    h = _dot(x, wgat_ref[0:NF, :]) + _dot(hln, wgat_ref[NF:D, :])
    esd = _dot(h, av_ref[...])
    es = esd[:, 0:1]
    ed = esd[:, 1:2]
    es_ref[...] = es
    ed_ref[...] = ed
    t = es + ed
    wself_ref[...] = jnp.exp(jnp.where(t >= 0.0, t, 0.2 * t))
    nb = hp_ref.shape[0]
    hp_ref[:, 0:D] = h
    hp_ref[:, D:D + 1] = jnp.ones((nb, 1), _f32)
    hp_ref[:, D + 1:DP] = jnp.zeros((nb, DP - D - 1), _f32)


def _fin_body(acc_ref, wself_ref, hp_ref, bgat_ref, xgat_ref):
    su = acc_ref[0] + acc_ref[1]
    h = hp_ref[:, 0:D]
    ws = wself_ref[...]
    num = su[:, 0:D] + ws * h
    den = su[:, D:D + 1] + ws
    xgat_ref[...] = num / den + bgat_ref[...]


def _gind_body(agg_ref, xgat_ref, eps_ref, w1_ref, b1_ref, w2_ref, b2_ref,
               xnext_ref):
    ag = agg_ref[0] + agg_ref[1]
    z = (1.0 + eps_ref[...]) * xgat_ref[...] + ag
    hmid = jnp.maximum(_dot(z, w1_ref[...]) + b1_ref[...], 0.0)
    xnext_ref[...] = _dot(hmid, w2_ref[...]) + b2_ref[...]


def _final_body(x_ref, pm_ref, cnt_ref, wg1_ref, bg1_ref, wg2_ref, bg2_ref,
                wf1_ref, bf1_ref, wf2_ref, bf2_ref, out_ref):
    mean = _dot(pm_ref[...], x_ref[...])
    g = jnp.maximum(_dot(mean, wg1_ref[...]) + bg1_ref[...], 0.0)
    hl = _dot(g, wg2_ref[...]) + bg2_ref[...]
    xg = mean * cnt_ref[...]
    t = jnp.maximum(
        _dot(xg, wf1_ref[0:NF, :]) + _dot(hl, wf1_ref[NF:D, :]) + bf1_ref[...],
        0.0)
    out_ref[...] = _dot(t, wf2_ref[...]) + bf2_ref[...]


def _tc(body, out_shapes):
    return pl.pallas_call(body, out_shape=out_shapes)


def kernel(x, edge_index, batchidxs, W_gat, a_src, a_dst, b_gat, eps,
           W1, b1, W2, b2, Wg1, bg1, Wg2, bg2, Wf1, bf1, Wf2, bf2):
    src = edge_index[0]
    dst = edge_index[1]

    onehot = (batchidxs[:, None] == jnp.arange(B, dtype=batchidxs.dtype)[None, :])
    rm = onehot.astype(_f32)
    cnt = jnp.sum(rm, axis=0)
    pm = rm.T * (1.0 / jnp.clip(cnt, 1.0))[:, None]
    cnt2 = cnt[:, None]

    av = jnp.stack([a_src, a_dst], axis=1)
    bg1r = bg1[None, :]
    bg2r = bg2[None, :]
    b1r = b1[None, :]
    b2r = b2[None, :]
    bf1r = bf1[None, :]
    bf2r = bf2[None, :]
    bgatr = b_gat[None, :]
    epsr = eps.reshape(1, 1)
    z160 = jnp.zeros((NSL, DP), _f32)
    z144 = jnp.zeros((NSL, D), _f32)

    FB = 10
    RB = N // FB
    hlvs = _tc(_hlvs_body, jax.ShapeDtypeStruct((B, NG), _f32))
    proj = pl.pallas_call(
        _proj_body,
        grid=(FB,),
        in_specs=[
            pl.BlockSpec((RB, NF), lambda i: (i, 0)),
            pl.BlockSpec((RB, B), lambda i: (i, 0)),
            pl.BlockSpec((B, NG), lambda i: (0, 0)),
            pl.BlockSpec((D, D), lambda i: (0, 0)),
            pl.BlockSpec((D, 2), lambda i: (0, 0)),
        ],
        out_specs=[
            pl.BlockSpec((RB, DP), lambda i: (i, 0)),
            pl.BlockSpec((RB, 1), lambda i: (i, 0)),
            pl.BlockSpec((RB, 1), lambda i: (i, 0)),
            pl.BlockSpec((RB, 1), lambda i: (i, 0)),
        ],
        out_shape=[
            jax.ShapeDtypeStruct((N, DP), _f32),
            jax.ShapeDtypeStruct((N, 1), _f32),
            jax.ShapeDtypeStruct((N, 1), _f32),
            jax.ShapeDtypeStruct((N, 1), _f32),
        ],
    )
    fin = pl.pallas_call(
        _fin_body,
        grid=(FB,),
        in_specs=[
            pl.BlockSpec((2, RB, DP), lambda i: (0, i, 0)),
            pl.BlockSpec((RB, 1), lambda i: (i, 0)),
            pl.BlockSpec((RB, DP), lambda i: (i, 0)),
            pl.BlockSpec((1, D), lambda i: (0, 0)),
        ],
        out_specs=pl.BlockSpec((RB, D), lambda i: (i, 0)),
        out_shape=jax.ShapeDtypeStruct((N, D), _f32),
    )
    gind = pl.pallas_call(
        _gind_body,
        grid=(FB,),
        in_specs=[
            pl.BlockSpec((2, RB, D), lambda i: (0, i, 0)),
            pl.BlockSpec((RB, D), lambda i: (i, 0)),
            pl.BlockSpec((1, 1), lambda i: (0, 0)),
            pl.BlockSpec((D, NH), lambda i: (0, 0)),
            pl.BlockSpec((1, NH), lambda i: (0, 0)),
            pl.BlockSpec((NH, NF), lambda i: (0, 0)),
            pl.BlockSpec((1, NF), lambda i: (0, 0)),
        ],
        out_specs=pl.BlockSpec((RB, NF), lambda i: (i, 0)),
        out_shape=jax.ShapeDtypeStruct((N, NF), _f32),
    )
    finl = _tc(_final_body, jax.ShapeDtypeStruct((B, 1), _f32))

    xc = x
    for _ in range(NPROP):
        hl = hlvs(xc, pm, Wg1, bg1r, Wg2, bg2r)
        hp, es, ed, wself = proj(xc, rm, hl, W_gat, av)
        acc = _gat_edges(hp, es.reshape(N), ed.reshape(N), src, dst, z160)
        xgat = fin(acc.reshape(2, N, DP), wself, hp, bgatr)
        agg = _gin_edges(xgat, src, dst, z144)
        xc = gind(agg.reshape(2, N, D), xgat, epsr, W1, b1r, W2, b2r)

    return finl(xc, pm, cnt2, Wg1, bg1r, Wg2, bg2r, Wf1, bf1r, Wf2, bf2r)


# speedup vs baseline: 16.2263x; 1.2826x over previous
import functools

import jax
import jax.numpy as jnp
from jax import lax
from jax.experimental import pallas as pl
from jax.experimental.pallas import tpu as pltpu
from jax.experimental.pallas import tpu_sc as plsc

N = 10000
E = 320000
B = 100
NF = 128
NG = 16
NH = 256
D = NF + NG
DP = 160
NPROP = 2

NC = 2
NS = 16
NW = NC * NS
EPW = E // NW
CH = 80
NSL = N // NS

_f32 = jnp.float32


def _mesh():
    return plsc.VectorSubcoreMesh(
        core_axis_name="c", subcore_axis_name="s", num_cores=NC, num_subcores=NS
    )


@functools.partial(
    pl.kernel,
    out_type=jax.ShapeDtypeStruct((NC * N, DP), _f32),
    mesh=_mesh(),
    scratch_types=[
        pltpu.VMEM((CH,), jnp.int32),
        pltpu.VMEM((CH,), jnp.int32),
        pltpu.VMEM((CH,), jnp.int32),
        pltpu.VMEM((CH,), jnp.int32),
        pltpu.VMEM((CH,), _f32),
        pltpu.VMEM((CH,), _f32),
        pltpu.VMEM((CH,), _f32),
        pltpu.VMEM((CH,), _f32),
        pltpu.VMEM((CH,), _f32),
        pltpu.VMEM((CH, DP), _f32),
        pltpu.VMEM((CH, DP), _f32),
        pltpu.VMEM_SHARED((N, DP), _f32),
        pltpu.SemaphoreType.DMA,
        pltpu.SemaphoreType.DMA,
        pltpu.SemaphoreType.DMA,
        pltpu.SemaphoreType.DMA,
    ],
    compiler_params=pltpu.CompilerParams(
        use_tc_tiling_on_sc=False, needs_layout_passes=False),
)
def _gat_edges(hp_hbm, es_hbm, ed_hbm, src_hbm, dst_hbm, z_hbm, out_hbm,
               idx_sa, idx_da, idx_sb, idx_db, sv_a, dv_a, sv_b, dv_b, w,
               rows_a, rows_b, acc, sem1a, sem2a, sem1b, sem2b):
    c = lax.axis_index("c")
    s = lax.axis_index("s")
    wid = c * NS + s
    ebase = wid * EPW

    pltpu.sync_copy(z_hbm, acc.at[pl.ds(s * NSL, NSL)])
    plsc.subcore_barrier()

    def halfchunk(idx_s, idx_d, sv, dv, rows):
        def grp(j, _):
            ev = sv[pl.ds(j * 16, 16)] + dv[pl.ds(j * 16, 16)]
            ev = jnp.where(ev >= 0.0, ev, 0.2 * ev)
            w[pl.ds(j * 16, 16)] = jnp.exp(ev)
            return 0

        lax.fori_loop(0, CH // 16, grp, 0)

        def rw(g, _):
            wv16 = w[pl.ds(g * 16, 16)]
            for l in range(16):
                wv = wv16[l]
                j = g * 16 + l
                for k in range(DP // 16):
                    rows[j, pl.ds(k * 16, 16)] = rows[j, pl.ds(k * 16, 16)] * wv
            return 0

        lax.fori_loop(0, CH // 16, rw, 0)
        pltpu.sync_copy(rows, acc.at[idx_d], add=True)

    def chunk_body(i, _):
        base_a = ebase + (2 * i) * CH
        base_b = base_a + CH
        pltpu.sync_copy(src_hbm.at[pl.ds(base_a, CH)], idx_sa)
        pltpu.sync_copy(dst_hbm.at[pl.ds(base_a, CH)], idx_da)
        cpra = pltpu.async_copy(hp_hbm.at[idx_sa], rows_a, sem1a)
        cpsa = pltpu.async_copy(es_hbm.at[idx_sa], sv_a, sem2a)
        cpda = pltpu.async_copy(ed_hbm.at[idx_da], dv_a, sem2a)
        pltpu.sync_copy(src_hbm.at[pl.ds(base_b, CH)], idx_sb)
        pltpu.sync_copy(dst_hbm.at[pl.ds(base_b, CH)], idx_db)
        cprb = pltpu.async_copy(hp_hbm.at[idx_sb], rows_b, sem1b)
        cpsb = pltpu.async_copy(es_hbm.at[idx_sb], sv_b, sem2b)
        cpdb = pltpu.async_copy(ed_hbm.at[idx_db], dv_b, sem2b)
        cpsa.wait()
        cpda.wait()
        cpra.wait()
        halfchunk(idx_sa, idx_da, sv_a, dv_a, rows_a)
        cpsb.wait()
        cpdb.wait()
        cprb.wait()
        halfchunk(idx_sb, idx_db, sv_b, dv_b, rows_b)
        return 0

    NPAIR = EPW // (2 * CH)
    lax.fori_loop(0, NPAIR, chunk_body, 0)

    for t in range(EPW // CH - 2 * NPAIR):
        base = ebase + (2 * NPAIR + t) * CH
        pltpu.sync_copy(src_hbm.at[pl.ds(base, CH)], idx_sa)
        pltpu.sync_copy(dst_hbm.at[pl.ds(base, CH)], idx_da)
        cpr = pltpu.async_copy(hp_hbm.at[idx_sa], rows_a, sem1a)
        cps = pltpu.async_copy(es_hbm.at[idx_sa], sv_a, sem2a)
        cpd = pltpu.async_copy(ed_hbm.at[idx_da], dv_a, sem2a)
        cps.wait()
        cpd.wait()
        cpr.wait()
        halfchunk(idx_sa, idx_da, sv_a, dv_a, rows_a)

    plsc.subcore_barrier()
    pltpu.sync_copy(acc.at[pl.ds(s * NSL, NSL)],
                    out_hbm.at[pl.ds(c * N + s * NSL, NSL)])


@functools.partial(
    pl.kernel,
    out_type=jax.ShapeDtypeStruct((NC * N, D), _f32),
    mesh=_mesh(),
    scratch_types=[
        pltpu.VMEM((CH,), jnp.int32),
        pltpu.VMEM((CH,), jnp.int32),
        pltpu.VMEM((CH,), jnp.int32),
        pltpu.VMEM((CH,), jnp.int32),
        pltpu.VMEM((CH, D), _f32),
        pltpu.VMEM((CH, D), _f32),
        pltpu.VMEM_SHARED((N, D), _f32),
        pltpu.SemaphoreType.DMA,
        pltpu.SemaphoreType.DMA,
    ],
    compiler_params=pltpu.CompilerParams(
        use_tc_tiling_on_sc=False, needs_layout_passes=False),
)
def _gin_edges(x_hbm, src_hbm, dst_hbm, z_hbm, out_hbm,
               idx_sa, idx_da, idx_sb, idx_db, rows_a, rows_b, acc,
               sem_a, sem_b):
    c = lax.axis_index("c")
    s = lax.axis_index("s")
    wid = c * NS + s
    ebase = wid * EPW

    pltpu.sync_copy(z_hbm, acc.at[pl.ds(s * NSL, NSL)])
    plsc.subcore_barrier()

    def chunk_body(i, _):
        base_a = ebase + (2 * i) * CH
        base_b = base_a + CH
        pltpu.sync_copy(src_hbm.at[pl.ds(base_a, CH)], idx_sa)
        pltpu.sync_copy(dst_hbm.at[pl.ds(base_a, CH)], idx_da)
        cpa = pltpu.async_copy(x_hbm.at[idx_sa], rows_a, sem_a)
        pltpu.sync_copy(src_hbm.at[pl.ds(base_b, CH)], idx_sb)
        pltpu.sync_copy(dst_hbm.at[pl.ds(base_b, CH)], idx_db)
        cpb = pltpu.async_copy(x_hbm.at[idx_sb], rows_b, sem_b)
        cpa.wait()
        pltpu.sync_copy(rows_a, acc.at[idx_da], add=True)
        cpb.wait()
        pltpu.sync_copy(rows_b, acc.at[idx_db], add=True)
        return 0

    NPAIR = EPW // (2 * CH)
    lax.fori_loop(0, NPAIR, chunk_body, 0)

    for t in range(EPW // CH - 2 * NPAIR):
        base = ebase + (2 * NPAIR + t) * CH
        pltpu.sync_copy(src_hbm.at[pl.ds(base, CH)], idx_sa)
        pltpu.sync_copy(dst_hbm.at[pl.ds(base, CH)], idx_da)
        pltpu.async_copy(x_hbm.at[idx_sa], rows_a, sem_a).wait()
        pltpu.sync_copy(rows_a, acc.at[idx_da], add=True)

    plsc.subcore_barrier()
    pltpu.sync_copy(acc.at[pl.ds(s * NSL, NSL)],
                    out_hbm.at[pl.ds(c * N + s * NSL, NSL)])


def _dot(a, b):
    return jnp.dot(a, b, precision=lax.Precision.HIGHEST,
                   preferred_element_type=_f32)


def _dot_hi(a, b):
    return jnp.dot(a, b, precision=lax.Precision.HIGHEST,
                   preferred_element_type=_f32)


def _hlvs_body(x_ref, pm_ref, wg1_ref, bg1_ref, wg2_ref, bg2_ref, hl_ref):
    mean = _dot_hi(pm_ref[...], x_ref[...])
    g = jnp.maximum(_dot(mean, wg1_ref[...]) + bg1_ref[...], 0.0)
    hl_ref[...] = _dot(g, wg2_ref[...]) + bg2_ref[...]


def _proj_body(x_ref, rm_ref, hl_ref, wgat_ref, av_ref,
               hp_ref, es_ref, ed_ref, wself_ref):
    x = x_ref[...]
    hln = _dot_hi(rm_ref[...], hl_ref[...])
    h = _dot(x, wgat_ref[0:NF, :]) + _dot(hln, wgat_ref[NF:D, :])
    esd = _dot(h, av_ref[...])
    es = esd[:, 0:1]
    ed = esd[:, 1:2]
    es_ref[...] = es
    ed_ref[...] = ed
    t = es + ed
    wself_ref[...] = jnp.exp(jnp.where(t >= 0.0, t, 0.2 * t))
    nb = hp_ref.shape[0]
    hp_ref[:, 0:D] = h
    hp_ref[:, D:D + 1] = jnp.ones((nb, 1), _f32)
    hp_ref[:, D + 1:DP] = jnp.zeros((nb, DP - D - 1), _f32)


def _fin_body(acc_ref, wself_ref, hp_ref, bgat_ref, xgat_ref):
    su = acc_ref[0] + acc_ref[1]
    h = hp_ref[:, 0:D]
    ws = wself_ref[...]
    num = su[:, 0:D] + ws * h
    den = su[:, D:D + 1] + ws
    xgat_ref[...] = num / den + bgat_ref[...]


def _gind_body(agg_ref, xgat_ref, eps_ref, w1_ref, b1_ref, w2_ref, b2_ref,
               xnext_ref):
    ag = agg_ref[0] + agg_ref[1]
    z = (1.0 + eps_ref[...]) * xgat_ref[...] + ag
    hmid = jnp.maximum(_dot(z, w1_ref[...]) + b1_ref[...], 0.0)
    xnext_ref[...] = _dot(hmid, w2_ref[...]) + b2_ref[...]


def _final_body(x_ref, pm_ref, cnt_ref, wg1_ref, bg1_ref, wg2_ref, bg2_ref,
                wf1_ref, bf1_ref, wf2_ref, bf2_ref, out_ref):
    mean = _dot(pm_ref[...], x_ref[...])
    g = jnp.maximum(_dot(mean, wg1_ref[...]) + bg1_ref[...], 0.0)
    hl = _dot(g, wg2_ref[...]) + bg2_ref[...]
    xg = mean * cnt_ref[...]
    t = jnp.maximum(
        _dot(xg, wf1_ref[0:NF, :]) + _dot(hl, wf1_ref[NF:D, :]) + bf1_ref[...],
        0.0)
    out_ref[...] = _dot(t, wf2_ref[...]) + bf2_ref[...]


def _tc(body, out_shapes):
    return pl.pallas_call(body, out_shape=out_shapes)


def kernel(x, edge_index, batchidxs, W_gat, a_src, a_dst, b_gat, eps,
           W1, b1, W2, b2, Wg1, bg1, Wg2, bg2, Wf1, bf1, Wf2, bf2):
    src = edge_index[0]
    dst = edge_index[1]

    onehot = (batchidxs[:, None] == jnp.arange(B, dtype=batchidxs.dtype)[None, :])
    rm = onehot.astype(_f32)
    cnt = jnp.sum(rm, axis=0)
    pm = rm.T * (1.0 / jnp.clip(cnt, 1.0))[:, None]
    cnt2 = cnt[:, None]

    av = jnp.stack([a_src, a_dst], axis=1)
    bg1r = bg1[None, :]
    bg2r = bg2[None, :]
    b1r = b1[None, :]
    b2r = b2[None, :]
    bf1r = bf1[None, :]
    bf2r = bf2[None, :]
    bgatr = b_gat[None, :]
    epsr = eps.reshape(1, 1)
    z160 = jnp.zeros((NSL, DP), _f32)
    z144 = jnp.zeros((NSL, D), _f32)

    FB = 10
    RB = N // FB
    hlvs = _tc(_hlvs_body, jax.ShapeDtypeStruct((B, NG), _f32))
    proj = pl.pallas_call(
        _proj_body,
        grid=(FB,),
        in_specs=[
            pl.BlockSpec((RB, NF), lambda i: (i, 0)),
            pl.BlockSpec((RB, B), lambda i: (i, 0)),
            pl.BlockSpec((B, NG), lambda i: (0, 0)),
            pl.BlockSpec((D, D), lambda i: (0, 0)),
            pl.BlockSpec((D, 2), lambda i: (0, 0)),
        ],
        out_specs=[
            pl.BlockSpec((RB, DP), lambda i: (i, 0)),
            pl.BlockSpec((RB, 1), lambda i: (i, 0)),
            pl.BlockSpec((RB, 1), lambda i: (i, 0)),
            pl.BlockSpec((RB, 1), lambda i: (i, 0)),
        ],
        out_shape=[
            jax.ShapeDtypeStruct((N, DP), _f32),
            jax.ShapeDtypeStruct((N, 1), _f32),
            jax.ShapeDtypeStruct((N, 1), _f32),
            jax.ShapeDtypeStruct((N, 1), _f32),
        ],
    )
    fin = pl.pallas_call(
        _fin_body,
        grid=(FB,),
        in_specs=[
            pl.BlockSpec((2, RB, DP), lambda i: (0, i, 0)),
            pl.BlockSpec((RB, 1), lambda i: (i, 0)),
            pl.BlockSpec((RB, DP), lambda i: (i, 0)),
            pl.BlockSpec((1, D), lambda i: (0, 0)),
        ],
        out_specs=pl.BlockSpec((RB, D), lambda i: (i, 0)),
        out_shape=jax.ShapeDtypeStruct((N, D), _f32),
    )
    gind = pl.pallas_call(
        _gind_body,
        grid=(FB,),
        in_specs=[
            pl.BlockSpec((2, RB, D), lambda i: (0, i, 0)),
            pl.BlockSpec((RB, D), lambda i: (i, 0)),
            pl.BlockSpec((1, 1), lambda i: (0, 0)),
            pl.BlockSpec((D, NH), lambda i: (0, 0)),
            pl.BlockSpec((1, NH), lambda i: (0, 0)),
            pl.BlockSpec((NH, NF), lambda i: (0, 0)),
            pl.BlockSpec((1, NF), lambda i: (0, 0)),
        ],
        out_specs=pl.BlockSpec((RB, NF), lambda i: (i, 0)),
        out_shape=jax.ShapeDtypeStruct((N, NF), _f32),
    )
    finl = _tc(_final_body, jax.ShapeDtypeStruct((B, 1), _f32))

    xc = x
    for _ in range(NPROP):
        hl = hlvs(xc, pm, Wg1, bg1r, Wg2, bg2r)
        hp, es, ed, wself = proj(xc, rm, hl, W_gat, av)
        acc = _gat_edges(hp, es.reshape(N), ed.reshape(N), src, dst, z160)
        xgat = fin(acc.reshape(2, N, DP), wself, hp, bgatr)
        agg = _gin_edges(xgat, src, dst, z144)
        xc = gind(agg.reshape(2, N, D), xgat, epsr, W1, b1r, W2, b2r)

    return finl(xc, pm, cnt2, Wg1, bg1r, Wg2, bg2r, Wf1, bf1r, Wf2, bf2r)
